# baseline (speedup 1.0000x reference)
import functools

import jax
import jax.numpy as jnp
from jax import lax
from jax.experimental import pallas as pl
from jax.experimental.pallas import tpu as pltpu
from jax.experimental.pallas import tpu_sc as plsc

_NUM_ENTRY = 1000
_E_PAD = 1024
_D = 256
_LOSS_SCALE = 0.1 + 0.2

_NC = 2
_NS = 16
_NW = _NC * _NS
_CHUNK = 128


def _vq_dist_block(f_ref, e_ref, idx_ref, loss_ref):
    f = f_ref[...]
    e = e_ref[...]
    scores2 = jax.lax.dot_general(
        f, e + e, (((1,), (1,)), ((), ())),
        preferred_element_type=jnp.float32,
        precision=jax.lax.Precision.DEFAULT)
    row = lax.broadcasted_iota(jnp.int32, (1, _E_PAD), 1)
    en = jnp.sum(e * e, axis=1)[None, :]
    en = jnp.where(row < _NUM_ENTRY, en, jnp.inf)
    fn = jnp.sum(f * f, axis=1, keepdims=True)
    dist = (fn + en) - scores2
    m = jnp.min(dist, axis=1, keepdims=True)
    col = lax.broadcasted_iota(jnp.int32, dist.shape, 1)
    idx = jnp.min(jnp.where(dist == m, col, _E_PAD), axis=1, keepdims=True)
    loss_ref[...] = m * (_LOSS_SCALE / _D)
    idx_ref[...] = idx


def _sc_gather_body(emb_hbm, idx_hbm, out_hbm,
                    idx_v, rows_a, rows_b, ga, gb, sa, sb):
    wid = lax.axis_index("s") * _NC + lax.axis_index("c")
    n_rows = out_hbm.shape[0]
    b_per_w = n_rows // _NW
    base = wid * b_per_w
    n_chunks = b_per_w // _CHUNK
    pltpu.sync_copy(idx_hbm.at[pl.ds(base, b_per_w)], idx_v)
    rows = (rows_a, rows_b)
    gsem = (ga, gb)
    ssem = (sa, sb)

    def gather_start(c):
        b = c % 2
        return pltpu.async_copy(
            emb_hbm.at[idx_v.at[pl.ds(c * _CHUNK, _CHUNK)]], rows[b], gsem[b])

    gath = [gather_start(0), None]
    scat = [None, None]
    for c in range(n_chunks):
        b = c % 2
        nb = (c + 1) % 2
        if c + 1 < n_chunks:
            if scat[nb] is not None:
                scat[nb].wait()
            gath[nb] = gather_start(c + 1)
        gath[b].wait()
        scat[b] = pltpu.async_copy(
            rows[b], out_hbm.at[pl.ds(base + c * _CHUNK, _CHUNK)], ssem[b])
    scat[0].wait()
    scat[1].wait()


def kernel(h, emb_weight):
    M = h.shape[0] * h.shape[1]
    flat = h.reshape(M, _D)
    e = jnp.pad(emb_weight, ((0, _E_PAD - _NUM_ENTRY), (0, 0)))
    BM = 4096
    NCHUNK = 4
    MC = M // NCHUNK
    bpc = MC // BM

    gather = functools.partial(
        pl.kernel,
        out_type=jax.ShapeDtypeStruct((MC, _D), jnp.float32),
        mesh=plsc.VectorSubcoreMesh(core_axis_name="c", subcore_axis_name="s"),
        scratch_types=[
            pltpu.VMEM((MC // _NW,), jnp.int32),
            pltpu.VMEM((_CHUNK, _D), jnp.float32),
            pltpu.VMEM((_CHUNK, _D), jnp.float32),
            pltpu.SemaphoreType.DMA,
            pltpu.SemaphoreType.DMA,
            pltpu.SemaphoreType.DMA,
            pltpu.SemaphoreType.DMA,
        ],
    )(_sc_gather_body)

    idxs, losses, qs = [], [], []
    for c in range(NCHUNK):
        idx_c, loss_c = pl.pallas_call(
            _vq_dist_block,
            grid=(bpc,),
            in_specs=[
                pl.BlockSpec((BM, _D), lambda i, c=c: (c * bpc + i, 0)),
                pl.BlockSpec((_E_PAD, _D), lambda i: (0, 0)),
            ],
            out_specs=[
                pl.BlockSpec((BM, 1), lambda i: (i, 0)),
                pl.BlockSpec((BM, 1), lambda i: (i, 0)),
            ],
            out_shape=[
                jax.ShapeDtypeStruct((MC, 1), jnp.int32),
                jax.ShapeDtypeStruct((MC, 1), jnp.float32),
            ],
        )(flat, e)
        idxs.append(idx_c)
        losses.append(loss_c)
        qs.append(gather(emb_weight, idx_c.reshape(MC)))
    idx = jnp.concatenate(idxs, axis=0)
    loss = jnp.concatenate(losses, axis=0)
    q = jnp.concatenate(qs, axis=0)
    return (q.reshape(h.shape), idx, loss[:, 0])


# speedup vs baseline: 1.3449x; 1.3449x over previous
import functools

import jax
import jax.numpy as jnp
from jax import lax
from jax.experimental import pallas as pl
from jax.experimental.pallas import tpu as pltpu
from jax.experimental.pallas import tpu_sc as plsc

_NUM_ENTRY = 1000
_E_PAD = 1024
_D = 256
_LOSS_SCALE = 0.1 + 0.2

_NC = 2
_NS = 16
_NW = _NC * _NS
_CHUNK = 128


def _vq_dist_block(f1_ref, f2_ref, e_ref, idx1_ref, idx2_ref,
                   loss1_ref, loss2_ref):
    e = e_ref[...]
    e2 = e + e
    row = lax.broadcasted_iota(jnp.int32, (1, _E_PAD), 1)
    en = jnp.sum(e * e, axis=1)[None, :]
    en = jnp.where(row < _NUM_ENTRY, en, jnp.inf)
    for f_ref, idx_ref, loss_ref in ((f1_ref, idx1_ref, loss1_ref),
                                     (f2_ref, idx2_ref, loss2_ref)):
        f = f_ref[...]
        scores2 = jax.lax.dot_general(
            f, e2, (((1,), (1,)), ((), ())),
            preferred_element_type=jnp.float32,
            precision=jax.lax.Precision.DEFAULT)
        fn = jnp.sum(f * f, axis=1, keepdims=True)
        dist = (fn + en) - scores2
        m = jnp.min(dist, axis=1, keepdims=True)
        col = lax.broadcasted_iota(jnp.int32, dist.shape, 1)
        idx = jnp.min(jnp.where(dist == m, col, _E_PAD), axis=1,
                      keepdims=True)
        loss_ref[...] = m * (_LOSS_SCALE / _D)
        idx_ref[...] = idx


def _sc_gather_body(emb_hbm, idx_hbm, out_hbm,
                    idx_v, rows_a, rows_b, ga, gb, sa, sb):
    wid = lax.axis_index("s") * _NC + lax.axis_index("c")
    n_rows = out_hbm.shape[0]
    b_per_w = n_rows // _NW
    base = wid * b_per_w
    n_chunks = b_per_w // _CHUNK
    pltpu.sync_copy(idx_hbm.at[pl.ds(base, b_per_w)], idx_v)
    rows = (rows_a, rows_b)
    gsem = (ga, gb)
    ssem = (sa, sb)

    def gather_start(c):
        b = c % 2
        return pltpu.async_copy(
            emb_hbm.at[idx_v.at[pl.ds(c * _CHUNK, _CHUNK)]], rows[b], gsem[b])

    gath = [gather_start(0), None]
    scat = [None, None]
    for c in range(n_chunks):
        b = c % 2
        nb = (c + 1) % 2
        if c + 1 < n_chunks:
            if scat[nb] is not None:
                scat[nb].wait()
            gath[nb] = gather_start(c + 1)
        gath[b].wait()
        scat[b] = pltpu.async_copy(
            rows[b], out_hbm.at[pl.ds(base + c * _CHUNK, _CHUNK)], ssem[b])
    scat[0].wait()
    scat[1].wait()


def kernel(h, emb_weight):
    M = h.shape[0] * h.shape[1]
    flat = h.reshape(M, _D)
    e = jnp.pad(emb_weight, ((0, _E_PAD - _NUM_ENTRY), (0, 0)))
    BM = 2048
    NB = M // BM
    HB = NB // 2
    MH = M // 2
    idx1, idx2, loss1, loss2 = pl.pallas_call(
        _vq_dist_block,
        grid=(HB,),
        in_specs=[
            pl.BlockSpec((BM, _D), lambda i: (i, 0)),
            pl.BlockSpec((BM, _D), lambda i: (HB + i, 0)),
            pl.BlockSpec((_E_PAD, _D), lambda i: (0, 0)),
        ],
        out_specs=[
            pl.BlockSpec((BM, 1), lambda i: (i, 0)),
            pl.BlockSpec((BM, 1), lambda i: (i, 0)),
            pl.BlockSpec((BM, 1), lambda i: (i, 0)),
            pl.BlockSpec((BM, 1), lambda i: (i, 0)),
        ],
        out_shape=[
            jax.ShapeDtypeStruct((MH, 1), jnp.int32),
            jax.ShapeDtypeStruct((MH, 1), jnp.int32),
            jax.ShapeDtypeStruct((MH, 1), jnp.float32),
            jax.ShapeDtypeStruct((MH, 1), jnp.float32),
        ],
    )(flat, flat, e)
    idx = jnp.concatenate([idx1, idx2], axis=0)
    loss = jnp.concatenate([loss1, loss2], axis=0)

    gather = functools.partial(
        pl.kernel,
        out_type=jax.ShapeDtypeStruct((M, _D), jnp.float32),
        mesh=plsc.VectorSubcoreMesh(core_axis_name="c", subcore_axis_name="s"),
        scratch_types=[
            pltpu.VMEM((M // _NW,), jnp.int32),
            pltpu.VMEM((_CHUNK, _D), jnp.float32),
            pltpu.VMEM((_CHUNK, _D), jnp.float32),
            pltpu.SemaphoreType.DMA,
            pltpu.SemaphoreType.DMA,
            pltpu.SemaphoreType.DMA,
            pltpu.SemaphoreType.DMA,
        ],
    )(_sc_gather_body)
    q = gather(emb_weight, idx.reshape(M))
    return (q.reshape(h.shape), idx, loss[:, 0])


# speedup vs baseline: 1.3511x; 1.0046x over previous
import functools

import jax
import jax.numpy as jnp
from jax import lax
from jax.experimental import pallas as pl
from jax.experimental.pallas import tpu as pltpu
from jax.experimental.pallas import tpu_sc as plsc

_NUM_ENTRY = 1000
_E_PAD = 1024
_D = 256
_LOSS_SCALE = 0.1 + 0.2

_NC = 2
_NS = 16
_NW = _NC * _NS
_CHUNK = 128


def _vq_dist_block(f1_ref, f2_ref, e_ref, idx1_ref, idx2_ref,
                   loss1_ref, loss2_ref):
    e = e_ref[...]
    e2 = e + e
    row = lax.broadcasted_iota(jnp.int32, (1, _E_PAD), 1)
    en = jnp.sum(e * e, axis=1)[None, :]
    en = jnp.where(row < _NUM_ENTRY, en, jnp.inf)
    for f_ref, idx_ref, loss_ref in ((f1_ref, idx1_ref, loss1_ref),
                                     (f2_ref, idx2_ref, loss2_ref)):
        f = f_ref[...]
        scores2 = jax.lax.dot_general(
            f, e2, (((1,), (1,)), ((), ())),
            preferred_element_type=jnp.float32,
            precision=jax.lax.Precision.DEFAULT)
        fn = jnp.sum(f * f, axis=1, keepdims=True)
        dist = (fn + en) - scores2
        m = jnp.min(dist, axis=1, keepdims=True)
        col = lax.broadcasted_iota(jnp.int32, dist.shape, 1)
        idx = jnp.min(jnp.where(dist == m, col, _E_PAD), axis=1,
                      keepdims=True)
        loss_ref[...] = m * (_LOSS_SCALE / _D)
        idx_ref[...] = idx


def _sc_gather_body(emb_hbm, idx_hbm, out_hbm,
                    idx_v, rows_a, rows_b, ga, gb, sa, sb):
    wid = lax.axis_index("s") * _NC + lax.axis_index("c")
    n_rows = out_hbm.shape[0]
    b_per_w = n_rows // _NW
    base = wid * b_per_w
    n_chunks = b_per_w // _CHUNK
    pltpu.sync_copy(idx_hbm.at[pl.ds(base, b_per_w)], idx_v)
    rows = (rows_a, rows_b)
    gsem = (ga, gb)
    ssem = (sa, sb)

    def gather_start(c):
        b = c % 2
        return pltpu.async_copy(
            emb_hbm.at[idx_v.at[pl.ds(c * _CHUNK, _CHUNK)]], rows[b], gsem[b])

    gath = [gather_start(0), None]
    scat = [None, None]
    for c in range(n_chunks):
        b = c % 2
        nb = (c + 1) % 2
        if c + 1 < n_chunks:
            if scat[nb] is not None:
                scat[nb].wait()
            gath[nb] = gather_start(c + 1)
        gath[b].wait()
        scat[b] = pltpu.async_copy(
            rows[b], out_hbm.at[pl.ds(base + c * _CHUNK, _CHUNK)], ssem[b])
    scat[0].wait()
    scat[1].wait()


def kernel(h, emb_weight):
    M = h.shape[0] * h.shape[1]
    flat = h.reshape(M, _D)
    e = jnp.pad(emb_weight, ((0, _E_PAD - _NUM_ENTRY), (0, 0)))
    BM = 4096
    NB = M // BM
    HB = NB // 2
    MH = M // 2
    idx1, idx2, loss1, loss2 = pl.pallas_call(
        _vq_dist_block,
        grid=(HB,),
        in_specs=[
            pl.BlockSpec((BM, _D), lambda i: (i, 0)),
            pl.BlockSpec((BM, _D), lambda i: (HB + i, 0)),
            pl.BlockSpec((_E_PAD, _D), lambda i: (0, 0)),
        ],
        out_specs=[
            pl.BlockSpec((BM, 1), lambda i: (i, 0)),
            pl.BlockSpec((BM, 1), lambda i: (i, 0)),
            pl.BlockSpec((BM, 1), lambda i: (i, 0)),
            pl.BlockSpec((BM, 1), lambda i: (i, 0)),
        ],
        out_shape=[
            jax.ShapeDtypeStruct((MH, 1), jnp.int32),
            jax.ShapeDtypeStruct((MH, 1), jnp.int32),
            jax.ShapeDtypeStruct((MH, 1), jnp.float32),
            jax.ShapeDtypeStruct((MH, 1), jnp.float32),
        ],
    )(flat, flat, e)
    idx = jnp.concatenate([idx1, idx2], axis=0)
    loss = jnp.concatenate([loss1, loss2], axis=0)

    gather = functools.partial(
        pl.kernel,
        out_type=jax.ShapeDtypeStruct((M, _D), jnp.float32),
        mesh=plsc.VectorSubcoreMesh(core_axis_name="c", subcore_axis_name="s"),
        scratch_types=[
            pltpu.VMEM((M // _NW,), jnp.int32),
            pltpu.VMEM((_CHUNK, _D), jnp.float32),
            pltpu.VMEM((_CHUNK, _D), jnp.float32),
            pltpu.SemaphoreType.DMA,
            pltpu.SemaphoreType.DMA,
            pltpu.SemaphoreType.DMA,
            pltpu.SemaphoreType.DMA,
        ],
    )(_sc_gather_body)
    q = gather(emb_weight, idx.reshape(M))
    return (q.reshape(h.shape), idx, loss[:, 0])
